# baseline (speedup 1.0000x reference)
import functools

import jax
import jax.numpy as jnp
from jax import lax
from jax.experimental import pallas as pl
from jax.experimental.pallas import tpu as pltpu
from jax.experimental.pallas import tpu_sc as plsc

HID = 768
EPS = 1e-6
L = 16
NV = HID // L
NC = 2
NS = 16
NW = NC * NS
CHUNK = 64


def _lanesum(x):
    lane = lax.iota(jnp.int32, L)
    for m in (1, 2, 4, 8):
        x = x + x.at[lane ^ m].get(mode="promise_in_bounds")
    return x


def _rsqrt16(v):
    i = lax.bitcast_convert_type(v, jnp.int32)
    y = lax.bitcast_convert_type(jnp.int32(0x5F3759DF) - (i >> 1), jnp.float32)
    h = v * 0.5
    for _ in range(3):
        y = y * (1.5 - h * y * y)
    return y


@functools.cache
def _build(n_tokens, seq):
    rows_per_w = n_tokens // NW
    nchunks = rows_per_w // CHUNK
    mesh = plsc.VectorSubcoreMesh(core_axis_name="c", subcore_axis_name="s")

    @functools.partial(
        pl.kernel,
        mesh=mesh,
        out_type=jax.ShapeDtypeStruct((n_tokens, HID), jnp.float32),
        scratch_types=[
            pltpu.VMEM((rows_per_w,), jnp.int32),
            pltpu.VMEM((CHUNK, HID), jnp.float32),
            pltpu.VMEM((CHUNK, HID), jnp.float32),
            pltpu.VMEM((HID,), jnp.float32),
            pltpu.VMEM((HID,), jnp.float32),
            pltpu.SemaphoreType.DMA,
            pltpu.SemaphoreType.DMA,
        ],
    )
    def k(ids_hbm, word_hbm, pos_hbm, gamma_hbm, beta_hbm, out_hbm,
          idx_v, buf, pbuf, gv, bv, sem_in, sem_out):
        wid = lax.axis_index("s") * NC + lax.axis_index("c")
        base = wid * rows_per_w
        s0 = base % seq

        pltpu.sync_copy(ids_hbm.at[pl.ds(base, rows_per_w)], idx_v)
        pltpu.sync_copy(gamma_hbm, gv)
        pltpu.sync_copy(beta_hbm, bv)

        def chunk_body(c, carry):
            row0 = pl.multiple_of(c * CHUNK, CHUNK)
            pltpu.async_copy(pos_hbm.at[pl.ds(s0 + row0, CHUNK)], pbuf, sem_in)
            pltpu.async_copy(word_hbm.at[idx_v.at[pl.ds(row0, CHUNK)]], buf,
                             sem_in).wait()
            pltpu.make_async_copy(pos_hbm.at[pl.ds(s0 + row0, CHUNK)], pbuf,
                                  sem_in).wait()

            def row_body(r, rcarry):
                vsum = jnp.zeros((L,), jnp.float32)
                vsq = jnp.zeros((L,), jnp.float32)
                for j in range(NV):
                    sl = pl.ds(j * L, L)
                    x = buf[r, sl] + pbuf[r, sl]
                    buf[r, sl] = x
                    vsum = vsum + x
                    vsq = vsq + x * x
                mean_v = _lanesum(vsum) * (1.0 / HID)
                msq_v = _lanesum(vsq) * (1.0 / HID)
                var_v = msq_v - mean_v * mean_v
                istd = _rsqrt16(var_v + EPS)
                for j in range(NV):
                    sl = pl.ds(j * L, L)
                    x = buf[r, sl]
                    buf[r, sl] = (x - mean_v) * istd * gv[sl] + bv[sl]
                return rcarry

            lax.fori_loop(0, CHUNK, row_body, 0)
            pltpu.async_copy(buf, out_hbm.at[pl.ds(base + row0, CHUNK)],
                             sem_out).wait()
            return carry

        lax.fori_loop(0, nchunks, chunk_body, 0)

    return k


def kernel(input_ids, word_embeddings, position_embeddings, gamma, beta):
    b, s = input_ids.shape
    ids = input_ids.reshape(-1).astype(jnp.int32)
    out = _build(b * s, s)(ids, word_embeddings, position_embeddings, gamma, beta)
    return out.reshape(b, s, HID)
